# baseline (speedup 1.0000x reference)
import functools

import jax
import jax.numpy as jnp
from jax import lax
from jax.experimental import pallas as pl
from jax.experimental.pallas import tpu as pltpu
from jax.experimental.pallas import tpu_sc as plsc

NC = 2
NS = 16
NW = NC * NS
CHUNK = 128


def _mesh():
  return plsc.VectorSubcoreMesh(
      core_axis_name="c", subcore_axis_name="s", num_cores=NC, num_subcores=NS)


def _load_idx(idx_hbm, idx_v, c, s, rcom, rext, ec):
  wid = c * NS + s
  pltpu.sync_copy(idx_hbm.at[pl.ds(wid * rcom, rcom)], idx_v.at[pl.ds(0, rcom)])
  if rext:
    @pl.when(c == ec)
    def _():
      base = NW * rcom + s * rext
      pltpu.sync_copy(idx_hbm.at[pl.ds(base, rext)],
                      idx_v.at[pl.ds(rcom, rext)])


def _make_deg_kernel(rows, rcom, rext, ec, np_, w):
  zr = np_ // NS
  rmax = rcom + rext

  @functools.partial(
      pl.kernel,
      mesh=_mesh(),
      out_type=jax.ShapeDtypeStruct((NC, np_, w), jnp.float32),
      scratch_types=[
          pltpu.VMEM((rmax, CHUNK), jnp.int32),
          pltpu.VMEM((CHUNK, w), jnp.float32),
          pltpu.VMEM((zr, w), jnp.float32),
          pltpu.VMEM_SHARED((np_, w), jnp.float32),
          pltpu.SemaphoreType.DMA,
      ],
      compiler_params=pltpu.CompilerParams(use_tc_tiling_on_sc=False),
  )
  def deg_kernel(dstp, ones_pat, zeros, out, dst_v, ones_v, ztile, acc, sem):
    c = lax.axis_index("c")
    s = lax.axis_index("s")
    pltpu.sync_copy(zeros.at[pl.ds(s * zr, zr)], ztile)
    pltpu.sync_copy(ztile, acc.at[pl.ds(s * zr, zr)])
    pltpu.sync_copy(ones_pat, ones_v)
    _load_idx(dstp, dst_v, c, s, rcom, rext, ec)
    rpw = rcom + jnp.where(c == ec, rext, 0)
    plsc.subcore_barrier()

    def body(j, carry):
      pltpu.async_copy(ones_v, acc.at[dst_v.at[j]], sem, add=True)
      return carry

    lax.fori_loop(0, rpw, body, 0)

    def drain(j, carry):
      pltpu.make_async_copy(ones_v, acc.at[dst_v.at[j]], sem).wait()
      return carry

    lax.fori_loop(0, rpw, drain, 0)
    plsc.subcore_barrier()
    pltpu.sync_copy(acc.at[pl.ds(s * zr, zr)], out.at[c, pl.ds(s * zr, zr)])

  return deg_kernel


RING = 8
LEAD = 4


def _make_msg_kernel(rows, rcom, rext, ec, np_, n, w):
  zr = np_ // NS
  rmax = rcom + rext
  assert rcom % RING == 0 and rext % RING == 0 and rcom >= 2 * RING

  @functools.partial(
      pl.kernel,
      mesh=_mesh(),
      out_type=jax.ShapeDtypeStruct((NC, np_, w), jnp.float32),
      scratch_types=[
          pltpu.VMEM((rmax, CHUNK), jnp.int32),
          pltpu.VMEM((rmax, CHUNK), jnp.int32),
          pltpu.VMEM((RING, CHUNK, w), jnp.float32),
          pltpu.VMEM((zr, w), jnp.float32),
          pltpu.VMEM_SHARED((np_, w), jnp.float32),
          pltpu.SemaphoreType.DMA((RING,)),
          pltpu.SemaphoreType.DMA((RING,)),
      ],
      compiler_params=pltpu.CompilerParams(use_tc_tiling_on_sc=False),
  )
  def msg_kernel(srcp, dstp, g, zeros, out, src_v, dst_v, bufs, ztile, acc,
                 gsem, ssem):
    c = lax.axis_index("c")
    s = lax.axis_index("s")
    pltpu.sync_copy(zeros.at[pl.ds(s * zr, zr)], ztile)
    pltpu.sync_copy(ztile, acc.at[pl.ds(s * zr, zr)])
    _load_idx(srcp, src_v, c, s, rcom, rext, ec)
    _load_idx(dstp, dst_v, c, s, rcom, rext, ec)
    nchunk = rcom + jnp.where(c == ec, rext, 0)
    plsc.subcore_barrier()

    def g_start(j, r):
      pltpu.async_copy(g.at[src_v.at[j]], bufs.at[r], gsem.at[r])

    def g_wait(j, r):
      pltpu.make_async_copy(g.at[src_v.at[j]], bufs.at[r], gsem.at[r]).wait()

    def s_start(j, r):
      pltpu.async_copy(bufs.at[r], acc.at[dst_v.at[j]], ssem.at[r], add=True)

    def s_wait(j, r):
      pltpu.make_async_copy(bufs.at[r], acc.at[dst_v.at[j]], ssem.at[r]).wait()

    nring = nchunk // RING
    for r in range(LEAD):
      g_start(r, r)
    for r in range(RING):
      g_wait(r, r)
      s_start(r, r)
      rn = (r + LEAD) % RING
      if r >= RING - LEAD:
        s_wait(rn, rn)
      g_start(r + LEAD, rn)

    def ring_body(k, carry):
      base = k * RING
      for r in range(RING):
        j = base + r
        g_wait(j, r)
        s_start(j, r)
        rn = (r + LEAD) % RING
        s_wait(j + LEAD - RING, rn)
        g_start(j + LEAD, rn)
      return carry

    lax.fori_loop(1, nring - 1, ring_body, 0)

    base = (nring - 1) * RING
    for r in range(RING):
      j = base + r
      g_wait(j, r)
      s_start(j, r)
      if r < RING - LEAD:
        rn = (r + LEAD) % RING
        s_wait(j + LEAD - RING, rn)
        g_start(j + LEAD, rn)
    for r in range(RING):
      s_wait(base + r, r)

    plsc.subcore_barrier()
    pltpu.sync_copy(acc.at[pl.ds(s * zr, zr)], out.at[c, pl.ds(s * zr, zr)])

  return msg_kernel


def _tc_mm(x, w1, n):
  def body(x_ref, w_ref, h_ref):
    h_ref[...] = jnp.dot(x_ref[...], w_ref[...],
                         preferred_element_type=jnp.float32,
                         precision=lax.Precision.HIGHEST)

  h_dim = w1.shape[1]
  return pl.pallas_call(
      body,
      out_shape=jax.ShapeDtypeStruct((n, h_dim), jnp.float32),
  )(x, w1)


def _tc_scale(h1, deg_parts, n):
  def body(h_ref, d_ref, g_ref, dinv_ref):
    d = d_ref[...]
    deg = d[0, :n, :] + d[1, :n, :]
    deg = jnp.sum(deg, axis=1, keepdims=True) + 1.0
    dinv = lax.rsqrt(deg)
    g_ref[...] = h_ref[...] * dinv
    dinv_ref[...] = dinv

  h_dim = h1.shape[1]
  return pl.pallas_call(
      body,
      out_shape=[
          jax.ShapeDtypeStruct((n, h_dim), jnp.float32),
          jax.ShapeDtypeStruct((n, 1), jnp.float32),
      ],
  )(h1, deg_parts)


def _tc_mid(parts, g1, dinv, b1, w2, n):
  def body(p_ref, g_ref, dinv_ref, b_ref, w_ref, g2_ref):
    p = p_ref[...]
    acc = p[0, :n, :] + p[1, :n, :] + g_ref[...]
    out1 = jnp.maximum(acc * dinv_ref[...] + b_ref[...], 0.0)
    g2_ref[...] = jnp.dot(
        out1, w_ref[...], preferred_element_type=jnp.float32,
        precision=lax.Precision.HIGHEST) * dinv_ref[...]

  w_pad = w2.shape[1]
  return pl.pallas_call(
      body,
      out_shape=jax.ShapeDtypeStruct((n, w_pad), jnp.float32),
  )(parts, g1, dinv, b1, w2)


def _tc_final(parts, g2, dinv, b2, n, c_dim):
  def body(p_ref, g_ref, dinv_ref, b_ref, o_ref):
    p = p_ref[...]
    acc = p[0, :n, :c_dim] + p[1, :n, :c_dim] + g_ref[..., :c_dim]
    o_ref[...] = jax.nn.sigmoid(acc * dinv_ref[...] + b_ref[...])

  return pl.pallas_call(
      body,
      out_shape=jax.ShapeDtypeStruct((n, c_dim), jnp.float32),
  )(parts, g2, dinv, b2)


def kernel(x, edge_index, W1, b1, W2, b2):
  n, d = x.shape
  h_dim = W1.shape[1]
  c_dim = W2.shape[1]
  e = edge_index.shape[1]

  rows = -(-e // CHUNK)
  rows = -(-rows // (8 * NW)) * (8 * NW)
  ep = rows * CHUNK
  rpw = rows // NW
  rcom = (rpw * 2 * 56) // 160 // RING * RING
  rext = 2 * rpw - 2 * rcom
  ec = 0
  np_ = -(-(n + 1) // (8 * NS)) * (8 * NS)

  src = edge_index[0]
  dst = edge_index[1]
  pad = ep - e
  srcp = jnp.concatenate([src, jnp.zeros((pad,), jnp.int32)]).reshape(
      rows, CHUNK)
  dstp = jnp.concatenate([dst, jnp.full((pad,), n, jnp.int32)]).reshape(
      rows, CHUNK)

  h1 = _tc_mm(x, W1, n)

  dw = 8
  ones_pat = jnp.zeros((CHUNK, dw), jnp.float32).at[:, 0].set(1.0)
  deg_parts = _make_deg_kernel(rows, rcom, rext, ec, np_, dw)(
      dstp, ones_pat, jnp.zeros((np_, dw), jnp.float32))

  g1, dinv = _tc_scale(h1, deg_parts, n)

  parts1 = _make_msg_kernel(rows, rcom, rext, ec, np_, n, h_dim)(
      srcp, dstp, g1, jnp.zeros((np_, h_dim), jnp.float32))

  w_pad = 8
  w2p = jnp.concatenate(
      [W2, jnp.zeros((h_dim, w_pad - c_dim), jnp.float32)], axis=1)
  g2 = _tc_mid(parts1, g1, dinv, b1.reshape(1, h_dim), w2p, n)

  parts2 = _make_msg_kernel(rows, rcom, rext, ec, np_, n, w_pad)(
      srcp, dstp, g2, jnp.zeros((np_, w_pad), jnp.float32))

  return _tc_final(parts2, g2, dinv, b2.reshape(1, c_dim), n, c_dim)


# speedup vs baseline: 1.0651x; 1.0651x over previous
import functools

import jax
import jax.numpy as jnp
from jax import lax
from jax.experimental import pallas as pl
from jax.experimental.pallas import tpu as pltpu
from jax.experimental.pallas import tpu_sc as plsc

NC = 2
NS = 16
NW = NC * NS
CHUNK = 128


def _mesh():
  return plsc.VectorSubcoreMesh(
      core_axis_name="c", subcore_axis_name="s", num_cores=NC, num_subcores=NS)


def _load_idx(idx_hbm, idx_v, c, s, rcom, rext, ec):
  wid = c * NS + s
  pltpu.sync_copy(idx_hbm.at[pl.ds(wid * rcom, rcom)], idx_v.at[pl.ds(0, rcom)])
  if rext:
    @pl.when(c == ec)
    def _():
      base = NW * rcom + s * rext
      pltpu.sync_copy(idx_hbm.at[pl.ds(base, rext)],
                      idx_v.at[pl.ds(rcom, rext)])


def _make_deg_kernel(rows, rcom, rext, ec, np_, w):
  zr = np_ // NS
  rmax = rcom + rext

  @functools.partial(
      pl.kernel,
      mesh=_mesh(),
      out_type=jax.ShapeDtypeStruct((NC, np_, w), jnp.float32),
      scratch_types=[
          pltpu.VMEM((rmax, CHUNK), jnp.int32),
          pltpu.VMEM((CHUNK, w), jnp.float32),
          pltpu.VMEM((zr, w), jnp.float32),
          pltpu.VMEM_SHARED((np_, w), jnp.float32),
          pltpu.SemaphoreType.DMA,
      ],
      compiler_params=pltpu.CompilerParams(use_tc_tiling_on_sc=False),
  )
  def deg_kernel(dstp, ones_pat, zeros, out, dst_v, ones_v, ztile, acc, sem):
    c = lax.axis_index("c")
    s = lax.axis_index("s")
    pltpu.sync_copy(zeros.at[pl.ds(s * zr, zr)], ztile)
    pltpu.sync_copy(ztile, acc.at[pl.ds(s * zr, zr)])
    pltpu.sync_copy(ones_pat, ones_v)
    _load_idx(dstp, dst_v, c, s, rcom, rext, ec)
    rpw = rcom + (jnp.where(c == ec, rext, 0) if rext else 0)
    plsc.subcore_barrier()

    def body(j, carry):
      pltpu.async_copy(ones_v, acc.at[dst_v.at[j]], sem, add=True)
      return carry

    lax.fori_loop(0, rpw, body, 0)

    def drain(j, carry):
      pltpu.make_async_copy(ones_v, acc.at[dst_v.at[j]], sem).wait()
      return carry

    lax.fori_loop(0, rpw, drain, 0)
    plsc.subcore_barrier()
    pltpu.sync_copy(acc.at[pl.ds(s * zr, zr)], out.at[c, pl.ds(s * zr, zr)])

  return deg_kernel


RING = 8
LEAD = 4


def _make_msg_kernel(rows, rcom, rext, ec, np_, n, w):
  zr = np_ // NS
  rmax = rcom + rext
  assert rcom % RING == 0 and rext % RING == 0 and rcom >= 2 * RING

  @functools.partial(
      pl.kernel,
      mesh=_mesh(),
      out_type=jax.ShapeDtypeStruct((NC, np_, w), jnp.float32),
      scratch_types=[
          pltpu.VMEM((rmax, CHUNK), jnp.int32),
          pltpu.VMEM((rmax, CHUNK), jnp.int32),
          pltpu.VMEM((RING, CHUNK, w), jnp.float32),
          pltpu.VMEM((zr, w), jnp.float32),
          pltpu.VMEM_SHARED((np_, w), jnp.float32),
          pltpu.SemaphoreType.DMA((RING,)),
          pltpu.SemaphoreType.DMA((RING,)),
      ],
      compiler_params=pltpu.CompilerParams(use_tc_tiling_on_sc=False),
  )
  def msg_kernel(srcp, dstp, g, zeros, out, src_v, dst_v, bufs, ztile, acc,
                 gsem, ssem):
    c = lax.axis_index("c")
    s = lax.axis_index("s")
    pltpu.sync_copy(zeros.at[pl.ds(s * zr, zr)], ztile)
    pltpu.sync_copy(ztile, acc.at[pl.ds(s * zr, zr)])
    _load_idx(srcp, src_v, c, s, rcom, rext, ec)
    _load_idx(dstp, dst_v, c, s, rcom, rext, ec)
    nchunk = rcom + (jnp.where(c == ec, rext, 0) if rext else 0)
    plsc.subcore_barrier()

    def g_start(j, r):
      pltpu.async_copy(g.at[src_v.at[j]], bufs.at[r], gsem.at[r])

    def g_wait(j, r):
      pltpu.make_async_copy(g.at[src_v.at[j]], bufs.at[r], gsem.at[r]).wait()

    def s_start(j, r):
      pltpu.async_copy(bufs.at[r], acc.at[dst_v.at[j]], ssem.at[r], add=True)

    def s_wait(j, r):
      pltpu.make_async_copy(bufs.at[r], acc.at[dst_v.at[j]], ssem.at[r]).wait()

    nring = nchunk // RING
    for r in range(LEAD):
      g_start(r, r)
    for r in range(RING):
      g_wait(r, r)
      s_start(r, r)
      rn = (r + LEAD) % RING
      if r >= RING - LEAD:
        s_wait(rn, rn)
      g_start(r + LEAD, rn)

    def ring_body(k, carry):
      base = k * RING
      for r in range(RING):
        j = base + r
        g_wait(j, r)
        s_start(j, r)
        rn = (r + LEAD) % RING
        s_wait(j + LEAD - RING, rn)
        g_start(j + LEAD, rn)
      return carry

    lax.fori_loop(1, nring - 1, ring_body, 0)

    base = (nring - 1) * RING
    for r in range(RING):
      j = base + r
      g_wait(j, r)
      s_start(j, r)
      if r < RING - LEAD:
        rn = (r + LEAD) % RING
        s_wait(j + LEAD - RING, rn)
        g_start(j + LEAD, rn)
    for r in range(RING):
      s_wait(base + r, r)

    plsc.subcore_barrier()
    pltpu.sync_copy(acc.at[pl.ds(s * zr, zr)], out.at[c, pl.ds(s * zr, zr)])

  return msg_kernel


def _tc_prescale1(x, w1, deg_parts, n):
  def body(x_ref, w_ref, d_ref, g_ref, dinv_ref):
    d = d_ref[...]
    deg = d[0, :n, :] + d[1, :n, :]
    deg = jnp.sum(deg, axis=1, keepdims=True) + 1.0
    dinv = lax.rsqrt(deg)
    h = jnp.dot(x_ref[...], w_ref[...], preferred_element_type=jnp.float32,
                precision=lax.Precision.HIGHEST)
    g_ref[...] = h * dinv
    dinv_ref[...] = dinv

  h_dim = w1.shape[1]
  return pl.pallas_call(
      body,
      out_shape=[
          jax.ShapeDtypeStruct((n, h_dim), jnp.float32),
          jax.ShapeDtypeStruct((n, 1), jnp.float32),
      ],
  )(x, w1, deg_parts)


def _tc_mid(parts, g1, dinv, b1, w2, n):
  def body(p_ref, g_ref, dinv_ref, b_ref, w_ref, g2_ref):
    p = p_ref[...]
    acc = p[0, :n, :] + p[1, :n, :] + g_ref[...]
    out1 = jnp.maximum(acc * dinv_ref[...] + b_ref[...], 0.0)
    g2_ref[...] = jnp.dot(
        out1, w_ref[...], preferred_element_type=jnp.float32,
        precision=lax.Precision.HIGHEST) * dinv_ref[...]

  w_pad = w2.shape[1]
  return pl.pallas_call(
      body,
      out_shape=jax.ShapeDtypeStruct((n, w_pad), jnp.float32),
  )(parts, g1, dinv, b1, w2)


def _tc_final(parts, g2, dinv, b2, n, c_dim):
  def body(p_ref, g_ref, dinv_ref, b_ref, o_ref):
    p = p_ref[...]
    acc = p[0, :n, :c_dim] + p[1, :n, :c_dim] + g_ref[..., :c_dim]
    o_ref[...] = jax.nn.sigmoid(acc * dinv_ref[...] + b_ref[...])

  return pl.pallas_call(
      body,
      out_shape=jax.ShapeDtypeStruct((n, c_dim), jnp.float32),
  )(parts, g2, dinv, b2)


def kernel(x, edge_index, W1, b1, W2, b2):
  n, d = x.shape
  h_dim = W1.shape[1]
  c_dim = W2.shape[1]
  e = edge_index.shape[1]

  rows = -(-e // CHUNK)
  rows = -(-rows // (8 * NW)) * (8 * NW)
  ep = rows * CHUNK
  rpw = rows // NW
  rcom = rpw
  rext = 0
  ec = 0
  np_ = -(-(n + 1) // (8 * NS)) * (8 * NS)

  src = edge_index[0]
  dst = edge_index[1]
  pad = ep - e
  srcp = jnp.concatenate([src, jnp.zeros((pad,), jnp.int32)]).reshape(
      rows, CHUNK)
  dstp = jnp.concatenate([dst, jnp.full((pad,), n, jnp.int32)]).reshape(
      rows, CHUNK)

  dw = 8
  ones_pat = jnp.zeros((CHUNK, dw), jnp.float32).at[:, 0].set(1.0)
  deg_parts = _make_deg_kernel(rows, rcom, rext, ec, np_, dw)(
      dstp, ones_pat, jnp.zeros((np_, dw), jnp.float32))

  g1, dinv = _tc_prescale1(x, W1, deg_parts, n)

  parts1 = _make_msg_kernel(rows, rcom, rext, ec, np_, n, h_dim)(
      srcp, dstp, g1, jnp.zeros((np_, h_dim), jnp.float32))

  w_pad = 8
  w2p = jnp.concatenate(
      [W2, jnp.zeros((h_dim, w_pad - c_dim), jnp.float32)], axis=1)
  g2 = _tc_mid(parts1, g1, dinv, b1.reshape(1, h_dim), w2p, n)

  parts2 = _make_msg_kernel(rows, rcom, rext, ec, np_, n, w_pad)(
      srcp, dstp, g2, jnp.zeros((np_, w_pad), jnp.float32))

  return _tc_final(parts2, g2, dinv, b2.reshape(1, c_dim), n, c_dim)


# speedup vs baseline: 1.4491x; 1.3606x over previous
import functools

import jax
import jax.numpy as jnp
from jax import lax
from jax.experimental import pallas as pl
from jax.experimental.pallas import tpu as pltpu
from jax.experimental.pallas import tpu_sc as plsc

NC = 2
NS = 16
NW = NC * NS
CHUNK = 128


def _mesh():
  return plsc.VectorSubcoreMesh(
      core_axis_name="c", subcore_axis_name="s", num_cores=NC, num_subcores=NS)


def _load_idx(idx_hbm, idx_v, c, s, rcom, rext, ec):
  wid = c * NS + s
  pltpu.sync_copy(idx_hbm.at[pl.ds(wid * rcom, rcom)], idx_v.at[pl.ds(0, rcom)])
  if rext:
    @pl.when(c == ec)
    def _():
      base = NW * rcom + s * rext
      pltpu.sync_copy(idx_hbm.at[pl.ds(base, rext)],
                      idx_v.at[pl.ds(rcom, rext)])


def _make_deg_kernel(rows, rcom, rext, ec, np_, w):
  zr = np_ // NS
  rmax = rcom + rext

  @functools.partial(
      pl.kernel,
      mesh=_mesh(),
      out_type=jax.ShapeDtypeStruct((NC, np_, w), jnp.float32),
      scratch_types=[
          pltpu.VMEM((rmax, CHUNK), jnp.int32),
          pltpu.VMEM((CHUNK, w), jnp.float32),
          pltpu.VMEM((zr, w), jnp.float32),
          pltpu.VMEM_SHARED((np_, w), jnp.float32),
          pltpu.SemaphoreType.DMA,
      ],
      compiler_params=pltpu.CompilerParams(use_tc_tiling_on_sc=False),
  )
  def deg_kernel(dstp, ones_pat, zeros, out, dst_v, ones_v, ztile, acc, sem):
    c = lax.axis_index("c")
    s = lax.axis_index("s")
    pltpu.sync_copy(zeros.at[pl.ds(s * zr, zr)], ztile)
    pltpu.sync_copy(ztile, acc.at[pl.ds(s * zr, zr)])
    pltpu.sync_copy(ones_pat, ones_v)
    _load_idx(dstp, dst_v, c, s, rcom, rext, ec)
    rpw = rcom + (jnp.where(c == ec, rext, 0) if rext else 0)
    plsc.subcore_barrier()

    def body(j, carry):
      pltpu.async_copy(ones_v, acc.at[dst_v.at[j]], sem, add=True)
      return carry

    lax.fori_loop(0, rpw, body, 0)

    def drain(j, carry):
      pltpu.make_async_copy(ones_v, acc.at[dst_v.at[j]], sem).wait()
      return carry

    lax.fori_loop(0, rpw, drain, 0)
    plsc.subcore_barrier()
    pltpu.sync_copy(acc.at[pl.ds(s * zr, zr)], out.at[c, pl.ds(s * zr, zr)])

  return deg_kernel


RING = 8
LEAD = 4


def _make_msg_kernel(rows, rcom, rext, ec, np_, n, w):
  zr = np_ // NS
  rmax = rcom + rext
  assert rcom % RING == 0 and rext % RING == 0 and rcom >= 2 * RING

  @functools.partial(
      pl.kernel,
      mesh=_mesh(),
      out_type=jax.ShapeDtypeStruct((NC, np_, w), jnp.float32),
      scratch_types=[
          pltpu.VMEM((rmax, CHUNK), jnp.int32),
          pltpu.VMEM((rmax, CHUNK), jnp.int32),
          pltpu.VMEM((RING, CHUNK, w), jnp.float32),
          pltpu.VMEM((zr, w), jnp.float32),
          pltpu.VMEM_SHARED((np_, w), jnp.float32),
          pltpu.VMEM_SHARED((np_, w), jnp.float32),
          pltpu.SemaphoreType.DMA((RING,)),
          pltpu.SemaphoreType.DMA((RING,)),
      ],
      compiler_params=pltpu.CompilerParams(use_tc_tiling_on_sc=False),
  )
  def msg_kernel(srcp, dstp, g, zeros, out, src_v, dst_v, bufs, ztile, acc,
                 g_sp, gsem, ssem):
    c = lax.axis_index("c")
    s = lax.axis_index("s")
    pltpu.sync_copy(zeros.at[pl.ds(s * zr, zr)], ztile)
    pltpu.sync_copy(ztile, acc.at[pl.ds(s * zr, zr)])
    pltpu.sync_copy(g.at[pl.ds(s * zr, zr)], ztile)
    pltpu.sync_copy(ztile, g_sp.at[pl.ds(s * zr, zr)])
    _load_idx(srcp, src_v, c, s, rcom, rext, ec)
    _load_idx(dstp, dst_v, c, s, rcom, rext, ec)
    nchunk = rcom + (jnp.where(c == ec, rext, 0) if rext else 0)
    plsc.subcore_barrier()

    def g_start(j, r):
      pltpu.async_copy(g_sp.at[src_v.at[j]], bufs.at[r], gsem.at[r])

    def g_wait(j, r):
      pltpu.make_async_copy(g_sp.at[src_v.at[j]], bufs.at[r], gsem.at[r]).wait()

    def s_start(j, r):
      pltpu.async_copy(bufs.at[r], acc.at[dst_v.at[j]], ssem.at[r], add=True)

    def s_wait(j, r):
      pltpu.make_async_copy(bufs.at[r], acc.at[dst_v.at[j]], ssem.at[r]).wait()

    nring = nchunk // RING
    for r in range(LEAD):
      g_start(r, r)
    for r in range(RING):
      g_wait(r, r)
      s_start(r, r)
      rn = (r + LEAD) % RING
      if r >= RING - LEAD:
        s_wait(rn, rn)
      g_start(r + LEAD, rn)

    def ring_body(k, carry):
      base = k * RING
      for r in range(RING):
        j = base + r
        g_wait(j, r)
        s_start(j, r)
        rn = (r + LEAD) % RING
        s_wait(j + LEAD - RING, rn)
        g_start(j + LEAD, rn)
      return carry

    lax.fori_loop(1, nring - 1, ring_body, 0)

    base = (nring - 1) * RING
    for r in range(RING):
      j = base + r
      g_wait(j, r)
      s_start(j, r)
      if r < RING - LEAD:
        rn = (r + LEAD) % RING
        s_wait(j + LEAD - RING, rn)
        g_start(j + LEAD, rn)
    for r in range(RING):
      s_wait(base + r, r)

    plsc.subcore_barrier()
    pltpu.sync_copy(acc.at[pl.ds(s * zr, zr)], out.at[c, pl.ds(s * zr, zr)])

  return msg_kernel


def _tc_prescale1(x, w1, deg_parts, n):
  def body(x_ref, w_ref, d_ref, g_ref, dinv_ref):
    d = d_ref[...]
    deg = d[0, :n, :] + d[1, :n, :]
    deg = jnp.sum(deg, axis=1, keepdims=True) + 1.0
    dinv = lax.rsqrt(deg)
    h = jnp.dot(x_ref[...], w_ref[...], preferred_element_type=jnp.float32,
                precision=lax.Precision.HIGHEST)
    g_ref[...] = h * dinv
    dinv_ref[...] = dinv

  h_dim = w1.shape[1]
  return pl.pallas_call(
      body,
      out_shape=[
          jax.ShapeDtypeStruct((n, h_dim), jnp.float32),
          jax.ShapeDtypeStruct((n, 1), jnp.float32),
      ],
  )(x, w1, deg_parts)


def _tc_mid(parts, g1, dinv, b1, w2, n):
  def body(p_ref, g_ref, dinv_ref, b_ref, w_ref, g2_ref):
    p = p_ref[...]
    acc = p[0, :n, :] + p[1, :n, :] + g_ref[...]
    out1 = jnp.maximum(acc * dinv_ref[...] + b_ref[...], 0.0)
    g2_ref[...] = jnp.dot(
        out1, w_ref[...], preferred_element_type=jnp.float32,
        precision=lax.Precision.HIGHEST) * dinv_ref[...]

  w_pad = w2.shape[1]
  return pl.pallas_call(
      body,
      out_shape=jax.ShapeDtypeStruct((n, w_pad), jnp.float32),
  )(parts, g1, dinv, b1, w2)


def _tc_final(parts, g2, dinv, b2, n, c_dim):
  def body(p_ref, g_ref, dinv_ref, b_ref, o_ref):
    p = p_ref[...]
    acc = p[0, :n, :c_dim] + p[1, :n, :c_dim] + g_ref[..., :c_dim]
    o_ref[...] = jax.nn.sigmoid(acc * dinv_ref[...] + b_ref[...])

  return pl.pallas_call(
      body,
      out_shape=jax.ShapeDtypeStruct((n, c_dim), jnp.float32),
  )(parts, g2, dinv, b2)


def kernel(x, edge_index, W1, b1, W2, b2):
  n, d = x.shape
  h_dim = W1.shape[1]
  c_dim = W2.shape[1]
  e = edge_index.shape[1]

  rows = -(-e // CHUNK)
  rows = -(-rows // (8 * NW)) * (8 * NW)
  ep = rows * CHUNK
  rpw = rows // NW
  rcom = rpw
  rext = 0
  ec = 0
  np_ = -(-(n + 1) // (8 * NS)) * (8 * NS)

  src = edge_index[0]
  dst = edge_index[1]
  pad = ep - e
  srcp = jnp.concatenate([src, jnp.zeros((pad,), jnp.int32)]).reshape(
      rows, CHUNK)
  dstp = jnp.concatenate([dst, jnp.full((pad,), n, jnp.int32)]).reshape(
      rows, CHUNK)

  dw = 8
  ones_pat = jnp.zeros((CHUNK, dw), jnp.float32).at[:, 0].set(1.0)
  deg_parts = _make_deg_kernel(rows, rcom, rext, ec, np_, dw)(
      dstp, ones_pat, jnp.zeros((np_, dw), jnp.float32))

  g1, dinv = _tc_prescale1(x, W1, deg_parts, n)

  parts1 = _make_msg_kernel(rows, rcom, rext, ec, np_, n, h_dim)(
      srcp, dstp, jnp.pad(g1, ((0, np_ - n), (0, 0))),
      jnp.zeros((np_, h_dim), jnp.float32))

  w_pad = 8
  w2p = jnp.concatenate(
      [W2, jnp.zeros((h_dim, w_pad - c_dim), jnp.float32)], axis=1)
  g2 = _tc_mid(parts1, g1, dinv, b1.reshape(1, h_dim), w2p, n)

  parts2 = _make_msg_kernel(rows, rcom, rext, ec, np_, n, w_pad)(
      srcp, dstp, jnp.pad(g2, ((0, np_ - n), (0, 0))),
      jnp.zeros((np_, w_pad), jnp.float32))

  return _tc_final(parts2, g2, dinv, b2.reshape(1, c_dim), n, c_dim)


# speedup vs baseline: 1.4559x; 1.0047x over previous
import functools

import jax
import jax.numpy as jnp
from jax import lax
from jax.experimental import pallas as pl
from jax.experimental.pallas import tpu as pltpu
from jax.experimental.pallas import tpu_sc as plsc

NC = 2
NS = 16
NW = NC * NS
CHUNK = 128


def _mesh():
  return plsc.VectorSubcoreMesh(
      core_axis_name="c", subcore_axis_name="s", num_cores=NC, num_subcores=NS)


def _load_idx(idx_hbm, idx_v, c, s, rcom, rext, ec):
  wid = c * NS + s
  pltpu.sync_copy(idx_hbm.at[pl.ds(wid * rcom, rcom)], idx_v.at[pl.ds(0, rcom)])
  if rext:
    @pl.when(c == ec)
    def _():
      base = NW * rcom + s * rext
      pltpu.sync_copy(idx_hbm.at[pl.ds(base, rext)],
                      idx_v.at[pl.ds(rcom, rext)])


def _make_deg_kernel(rows, rcom, rext, ec, np_, w):
  zr = np_ // NS
  rmax = rcom + rext

  @functools.partial(
      pl.kernel,
      mesh=_mesh(),
      out_type=jax.ShapeDtypeStruct((NC, np_, w), jnp.float32),
      scratch_types=[
          pltpu.VMEM((rmax, CHUNK), jnp.int32),
          pltpu.VMEM((CHUNK, w), jnp.float32),
          pltpu.VMEM((zr, w), jnp.float32),
          pltpu.VMEM_SHARED((np_, w), jnp.float32),
          pltpu.SemaphoreType.DMA,
      ],
      compiler_params=pltpu.CompilerParams(use_tc_tiling_on_sc=False),
  )
  def deg_kernel(dstp, ones_pat, zeros, out, dst_v, ones_v, ztile, acc, sem):
    c = lax.axis_index("c")
    s = lax.axis_index("s")
    pltpu.sync_copy(zeros.at[pl.ds(s * zr, zr)], ztile)
    pltpu.sync_copy(ztile, acc.at[pl.ds(s * zr, zr)])
    pltpu.sync_copy(ones_pat, ones_v)
    _load_idx(dstp, dst_v, c, s, rcom, rext, ec)
    rpw = rcom + (jnp.where(c == ec, rext, 0) if rext else 0)
    plsc.subcore_barrier()

    def body(j, carry):
      pltpu.async_copy(ones_v, acc.at[dst_v.at[j]], sem, add=True)
      return carry

    lax.fori_loop(0, rpw, body, 0)

    def drain(j, carry):
      pltpu.make_async_copy(ones_v, acc.at[dst_v.at[j]], sem).wait()
      return carry

    lax.fori_loop(0, rpw, drain, 0)
    plsc.subcore_barrier()
    pltpu.sync_copy(acc.at[pl.ds(s * zr, zr)], out.at[c, pl.ds(s * zr, zr)])

  return deg_kernel


RING = 8
LEAD = 4


def _make_msg_kernel(rows, rcom, rext, ec, np_, n, w):
  zr = np_ // NS
  rmax = rcom + rext
  assert rcom % RING == 0 and rext % RING == 0 and rcom >= 2 * RING

  @functools.partial(
      pl.kernel,
      mesh=_mesh(),
      out_type=jax.ShapeDtypeStruct((NC, np_, w), jnp.float32),
      scratch_types=[
          pltpu.VMEM((rmax, CHUNK), jnp.int32),
          pltpu.VMEM((rmax, CHUNK), jnp.int32),
          pltpu.VMEM((RING, CHUNK, w), jnp.float32),
          pltpu.VMEM((zr, w), jnp.float32),
          pltpu.VMEM_SHARED((np_, w), jnp.float32),
          pltpu.VMEM_SHARED((np_, w), jnp.float32),
          pltpu.SemaphoreType.DMA((RING,)),
          pltpu.SemaphoreType.DMA((RING,)),
      ],
      compiler_params=pltpu.CompilerParams(use_tc_tiling_on_sc=False),
  )
  def msg_kernel(srcp, dstp, g, zeros, out, src_v, dst_v, bufs, ztile, acc,
                 g_sp, gsem, ssem):
    c = lax.axis_index("c")
    s = lax.axis_index("s")
    pltpu.sync_copy(zeros.at[pl.ds(s * zr, zr)], ztile)
    pltpu.sync_copy(ztile, acc.at[pl.ds(s * zr, zr)])
    pltpu.sync_copy(g.at[pl.ds(s * zr, zr)], ztile)
    pltpu.sync_copy(ztile, g_sp.at[pl.ds(s * zr, zr)])
    _load_idx(srcp, src_v, c, s, rcom, rext, ec)
    _load_idx(dstp, dst_v, c, s, rcom, rext, ec)
    nchunk = rcom + (jnp.where(c == ec, rext, 0) if rext else 0)
    plsc.subcore_barrier()

    def g_start(j, r):
      pltpu.async_copy(g_sp.at[src_v.at[j]], bufs.at[r], gsem.at[r])

    def g_wait(j, r):
      pltpu.make_async_copy(g_sp.at[src_v.at[j]], bufs.at[r], gsem.at[r]).wait()

    def s_start(j, r):
      pltpu.async_copy(bufs.at[r], acc.at[dst_v.at[j]], ssem.at[r], add=True)

    def s_wait(j, r):
      pltpu.make_async_copy(bufs.at[r], acc.at[dst_v.at[j]], ssem.at[r]).wait()

    nring = nchunk // RING
    for r in range(LEAD):
      g_start(r, r)
    for r in range(RING):
      g_wait(r, r)
      s_start(r, r)
      rn = (r + LEAD) % RING
      if r >= RING - LEAD:
        s_wait(rn, rn)
      g_start(r + LEAD, rn)

    def ring_body(k, carry):
      base = k * RING
      for r in range(RING):
        j = base + r
        g_wait(j, r)
        s_start(j, r)
        rn = (r + LEAD) % RING
        s_wait(j + LEAD - RING, rn)
        g_start(j + LEAD, rn)
      return carry

    lax.fori_loop(1, nring - 1, ring_body, 0)

    base = (nring - 1) * RING
    for r in range(RING):
      j = base + r
      g_wait(j, r)
      s_start(j, r)
      if r < RING - LEAD:
        rn = (r + LEAD) % RING
        s_wait(j + LEAD - RING, rn)
        g_start(j + LEAD, rn)
    for r in range(RING):
      s_wait(base + r, r)

    plsc.subcore_barrier()
    pltpu.sync_copy(acc.at[pl.ds(s * zr, zr)], out.at[c, pl.ds(s * zr, zr)])

  return msg_kernel


def _tc_prescale1(x, w1, deg_parts, n):
  def body(x_ref, w_ref, d_ref, g_ref, dinv_ref):
    d = d_ref[...]
    deg = d[0, :n, :] + d[1, :n, :]
    deg = jnp.sum(deg, axis=1, keepdims=True) + 1.0
    dinv = lax.rsqrt(deg)
    h = jnp.dot(x_ref[...], w_ref[...], preferred_element_type=jnp.float32,
                precision=lax.Precision.HIGHEST)
    g_ref[:n, :] = h * dinv
    g_ref[n:, :] = jnp.zeros_like(g_ref[n:, :])
    dinv_ref[...] = dinv

  h_dim = w1.shape[1]
  np_ = deg_parts.shape[1]
  return pl.pallas_call(
      body,
      out_shape=[
          jax.ShapeDtypeStruct((np_, h_dim), jnp.float32),
          jax.ShapeDtypeStruct((n, 1), jnp.float32),
      ],
  )(x, w1, deg_parts)


def _tc_mid(parts, g1, dinv, b1, w2, n):
  def body(p_ref, g_ref, dinv_ref, b_ref, w_ref, g2_ref):
    p = p_ref[...]
    acc = p[0, :n, :] + p[1, :n, :] + g_ref[:n, :]
    out1 = jnp.maximum(acc * dinv_ref[...] + b_ref[...], 0.0)
    g2_ref[:n, :] = jnp.dot(
        out1, w_ref[...], preferred_element_type=jnp.float32,
        precision=lax.Precision.HIGHEST) * dinv_ref[...]
    g2_ref[n:, :] = jnp.zeros_like(g2_ref[n:, :])

  w_pad = w2.shape[1]
  np_ = parts.shape[1]
  return pl.pallas_call(
      body,
      out_shape=jax.ShapeDtypeStruct((np_, w_pad), jnp.float32),
  )(parts, g1, dinv, b1, w2)


def _tc_final(parts, g2, dinv, b2, n, c_dim):
  def body(p_ref, g_ref, dinv_ref, b_ref, o_ref):
    p = p_ref[...]
    acc = p[0, :n, :c_dim] + p[1, :n, :c_dim] + g_ref[:n, :c_dim]
    o_ref[...] = jax.nn.sigmoid(acc * dinv_ref[...] + b_ref[...])

  return pl.pallas_call(
      body,
      out_shape=jax.ShapeDtypeStruct((n, c_dim), jnp.float32),
  )(parts, g2, dinv, b2)


def kernel(x, edge_index, W1, b1, W2, b2):
  n, d = x.shape
  h_dim = W1.shape[1]
  c_dim = W2.shape[1]
  e = edge_index.shape[1]

  rows = -(-e // CHUNK)
  rows = -(-rows // (8 * NW)) * (8 * NW)
  ep = rows * CHUNK
  rpw = rows // NW
  rcom = rpw
  rext = 0
  ec = 0
  np_ = -(-(n + 1) // (8 * NS)) * (8 * NS)

  src = edge_index[0]
  dst = edge_index[1]
  pad = ep - e
  srcp = jnp.concatenate([src, jnp.zeros((pad,), jnp.int32)]).reshape(
      rows, CHUNK)
  dstp = jnp.concatenate([dst, jnp.full((pad,), n, jnp.int32)]).reshape(
      rows, CHUNK)

  dw = 8
  ones_pat = jnp.zeros((CHUNK, dw), jnp.float32).at[:, 0].set(1.0)
  deg_parts = _make_deg_kernel(rows, rcom, rext, ec, np_, dw)(
      dstp, ones_pat, jnp.zeros((np_, dw), jnp.float32))

  g1, dinv = _tc_prescale1(x, W1, deg_parts, n)

  parts1 = _make_msg_kernel(rows, rcom, rext, ec, np_, n, h_dim)(
      srcp, dstp, g1, jnp.zeros((np_, h_dim), jnp.float32))

  w_pad = 8
  w2p = jnp.concatenate(
      [W2, jnp.zeros((h_dim, w_pad - c_dim), jnp.float32)], axis=1)
  g2 = _tc_mid(parts1, g1, dinv, b1.reshape(1, h_dim), w2p, n)

  parts2 = _make_msg_kernel(rows, rcom, rext, ec, np_, n, w_pad)(
      srcp, dstp, g2, jnp.zeros((np_, w_pad), jnp.float32))

  return _tc_final(parts2, g2, dinv, b2.reshape(1, c_dim), n, c_dim)
